# initial kernel scaffold (unmeasured)
import jax
import jax.numpy as jnp
from jax import lax
from jax.experimental import pallas as pl
from jax.experimental.pallas import tpu as pltpu


def kernel(
    u,
):
    def body(*refs):
        pass

    out_shape = jax.ShapeDtypeStruct(..., jnp.float32)
    return pl.pallas_call(body, out_shape=out_shape)(...)


# baseline (device time: 13486 ns/iter reference)
import jax
import jax.numpy as jnp
from jax import lax
from jax.experimental import pallas as pl
from jax.experimental.pallas import tpu as pltpu

NX, NY, NZ = 2, 4, 4

XLO, XHI, YLO, YHI, ZLO, ZHI = range(6)


def kernel(u):
    sx, sy, sz = u.shape

    def body(
        u_ref,
        out_ref,
        sxlo, sxhi, sylo, syhi, szlo, szhi,
        hxlo, hxhi, hylo, hyhi, hzlo, hzhi,
        send_sems, recv_sems,
    ):
        mx = lax.axis_index("x")
        my = lax.axis_index("y")
        mz = lax.axis_index("z")

        has_xlo = mx > 0
        has_xhi = mx < NX - 1
        has_ylo = my > 0
        has_yhi = my < NY - 1
        has_zlo = mz > 0
        has_zhi = mz < NZ - 1

        barrier = pltpu.get_barrier_semaphore()
        for cond, dev in (
            (has_xlo, (mx - 1, my, mz)),
            (has_xhi, (mx + 1, my, mz)),
            (has_ylo, (mx, my - 1, mz)),
            (has_yhi, (mx, my + 1, mz)),
            (has_zlo, (mx, my, mz - 1)),
            (has_zhi, (mx, my, mz + 1)),
        ):
            @pl.when(cond)
            def _(dev=dev):
                pl.semaphore_signal(
                    barrier, inc=1,
                    device_id=dev, device_id_type=pl.DeviceIdType.MESH,
                )
        for cond in (has_xlo, has_xhi, has_ylo, has_yhi, has_zlo, has_zhi):
            @pl.when(cond)
            def _():
                pl.semaphore_wait(barrier, 1)

        sxlo[:, :, :] = u_ref[0:1, :, :]
        sxhi[:, :, :] = u_ref[sx - 1:sx, :, :]
        sylo[:, :, :] = u_ref[:, 0:1, :]
        syhi[:, :, :] = u_ref[:, sy - 1:sy, :]
        szlo[:, :, :] = u_ref[:, :, 0:1]
        szhi[:, :, :] = u_ref[:, :, sz - 1:sz]

        def send(src, dst, sdir, rdir, dev):
            rd = pltpu.make_async_remote_copy(
                src_ref=src, dst_ref=dst,
                send_sem=send_sems.at[sdir], recv_sem=recv_sems.at[rdir],
                device_id=dev, device_id_type=pl.DeviceIdType.MESH,
            )
            rd.start()

        @pl.when(has_xlo)
        def _():
            send(sxlo, hxhi, XLO, XHI, (mx - 1, my, mz))

        @pl.when(has_xhi)
        def _():
            send(sxhi, hxlo, XHI, XLO, (mx + 1, my, mz))

        @pl.when(has_ylo)
        def _():
            send(sylo, hyhi, YLO, YHI, (mx, my - 1, mz))

        @pl.when(has_yhi)
        def _():
            send(syhi, hylo, YHI, YLO, (mx, my + 1, mz))

        @pl.when(has_zlo)
        def _():
            send(szlo, hzhi, ZLO, ZHI, (mx, my, mz - 1))

        @pl.when(has_zhi)
        def _():
            send(szhi, hzlo, ZHI, ZLO, (mx, my, mz + 1))

        def wait_halo(buf, rdir):
            rd = pltpu.make_async_remote_copy(
                src_ref=buf, dst_ref=buf,
                send_sem=send_sems.at[rdir], recv_sem=recv_sems.at[rdir],
                device_id=(mx, my, mz), device_id_type=pl.DeviceIdType.MESH,
            )
            rd.wait_recv()

        for cond, buf, rdir in (
            (has_xlo, hxlo, XLO),
            (has_xhi, hxhi, XHI),
            (has_ylo, hylo, YLO),
            (has_yhi, hyhi, YHI),
            (has_zlo, hzlo, ZLO),
            (has_zhi, hzhi, ZHI),
        ):
            @pl.when(cond)
            def _(buf=buf, rdir=rdir):
                wait_halo(buf, rdir)

        uc = u_ref[:, :, :]
        v = (
            jnp.concatenate([hxlo[:, :, :], uc[:-1, :, :]], axis=0)
            + jnp.concatenate([uc[1:, :, :], hxhi[:, :, :]], axis=0)
            + jnp.concatenate([hylo[:, :, :], uc[:, :-1, :]], axis=1)
            + jnp.concatenate([uc[:, 1:, :], hyhi[:, :, :]], axis=1)
            + jnp.concatenate([hzlo[:, :, :], uc[:, :, :-1]], axis=2)
            + jnp.concatenate([uc[:, :, 1:], hzhi[:, :, :]], axis=2)
            - 6.0 * uc
        )

        ix = lax.broadcasted_iota(jnp.int32, (sx, sy, sz), 0) + mx * sx
        iy = lax.broadcasted_iota(jnp.int32, (sx, sy, sz), 1) + my * sy
        iz = lax.broadcasted_iota(jnp.int32, (sx, sy, sz), 2) + mz * sz
        interior = (
            (ix > 0) & (ix < NX * sx - 1)
            & (iy > 0) & (iy < NY * sy - 1)
            & (iz > 0) & (iz < NZ * sz - 1)
        )
        out_ref[:, :, :] = jnp.where(interior, v, 0.0)

        def wait_sent(src, sdir):
            rd = pltpu.make_async_remote_copy(
                src_ref=src, dst_ref=src,
                send_sem=send_sems.at[sdir], recv_sem=recv_sems.at[sdir],
                device_id=(mx, my, mz), device_id_type=pl.DeviceIdType.MESH,
            )
            rd.wait_send()

        for cond, src, sdir in (
            (has_xlo, sxlo, XLO),
            (has_xhi, sxhi, XHI),
            (has_ylo, sylo, YLO),
            (has_yhi, syhi, YHI),
            (has_zlo, szlo, ZLO),
            (has_zhi, szhi, ZHI),
        ):
            @pl.when(cond)
            def _(src=src, sdir=sdir):
                wait_sent(src, sdir)

    face = lambda shape: pltpu.VMEM(shape, jnp.float32)
    return pl.pallas_call(
        body,
        out_shape=jax.ShapeDtypeStruct((sx, sy, sz), jnp.float32),
        in_specs=[pl.BlockSpec(memory_space=pltpu.VMEM)],
        out_specs=pl.BlockSpec(memory_space=pltpu.VMEM),
        scratch_shapes=[
            face((1, sy, sz)), face((1, sy, sz)),
            face((sx, 1, sz)), face((sx, 1, sz)),
            face((sx, sy, 1)), face((sx, sy, 1)),
            face((1, sy, sz)), face((1, sy, sz)),
            face((sx, 1, sz)), face((sx, 1, sz)),
            face((sx, sy, 1)), face((sx, sy, 1)),
            pltpu.SemaphoreType.DMA((6,)),
            pltpu.SemaphoreType.DMA((6,)),
        ],
        compiler_params=pltpu.CompilerParams(collective_id=0),
    )(u)


# device time: 11289 ns/iter; 1.1946x vs baseline; 1.1946x over previous
import jax
import jax.numpy as jnp
from jax import lax
from jax.experimental import pallas as pl
from jax.experimental.pallas import tpu as pltpu

NX, NY, NZ = 2, 4, 4

XLO, XHI, YLO, YHI, ZLO, ZHI = range(6)


def kernel(u):
    sx, sy, sz = u.shape

    def body(
        u_ref,
        out_ref,
        szlo, szhi,
        hxlo, hxhi, hylo, hyhi, hzlo, hzhi,
        send_sems, recv_sems,
    ):
        mx = lax.axis_index("x")
        my = lax.axis_index("y")
        mz = lax.axis_index("z")

        has_xlo = mx > 0
        has_xhi = mx < NX - 1
        has_ylo = my > 0
        has_yhi = my < NY - 1
        has_zlo = mz > 0
        has_zhi = mz < NZ - 1

        barrier = pltpu.get_barrier_semaphore()
        for cond, dev in (
            (has_xlo, (mx - 1, my, mz)),
            (has_xhi, (mx + 1, my, mz)),
            (has_ylo, (mx, my - 1, mz)),
            (has_yhi, (mx, my + 1, mz)),
            (has_zlo, (mx, my, mz - 1)),
            (has_zhi, (mx, my, mz + 1)),
        ):
            @pl.when(cond)
            def _(dev=dev):
                pl.semaphore_signal(
                    barrier, inc=1,
                    device_id=dev, device_id_type=pl.DeviceIdType.MESH,
                )
        for cond in (has_xlo, has_xhi, has_ylo, has_yhi, has_zlo, has_zhi):
            @pl.when(cond)
            def _():
                pl.semaphore_wait(barrier, 1)

        def send(src, dst, sdir, rdir, dev):
            rd = pltpu.make_async_remote_copy(
                src_ref=src, dst_ref=dst,
                send_sem=send_sems.at[sdir], recv_sem=recv_sems.at[rdir],
                device_id=dev, device_id_type=pl.DeviceIdType.MESH,
            )
            rd.start()

        @pl.when(has_xlo)
        def _():
            send(u_ref.at[0:1, :, :], hxhi, XLO, XHI, (mx - 1, my, mz))

        @pl.when(has_xhi)
        def _():
            send(u_ref.at[sx - 1:sx, :, :], hxlo, XHI, XLO, (mx + 1, my, mz))

        @pl.when(has_ylo)
        def _():
            send(u_ref.at[:, 0:1, :], hyhi, YLO, YHI, (mx, my - 1, mz))

        @pl.when(has_yhi)
        def _():
            send(u_ref.at[:, sy - 1:sy, :], hylo, YHI, YLO, (mx, my + 1, mz))

        szlo[:, :, :] = u_ref[:, :, 0:1]
        szhi[:, :, :] = u_ref[:, :, sz - 1:sz]

        @pl.when(has_zlo)
        def _():
            send(szlo, hzhi, ZLO, ZHI, (mx, my, mz - 1))

        @pl.when(has_zhi)
        def _():
            send(szhi, hzlo, ZHI, ZLO, (mx, my, mz + 1))

        uc = u_ref[:, :, :]
        zx = jnp.zeros((1, sy, sz), jnp.float32)
        zy = jnp.zeros((sx, 1, sz), jnp.float32)
        zz = jnp.zeros((sx, sy, 1), jnp.float32)
        v0 = (
            jnp.concatenate([zx, uc[:-1, :, :]], axis=0)
            + jnp.concatenate([uc[1:, :, :], zx], axis=0)
            + jnp.concatenate([zy, uc[:, :-1, :]], axis=1)
            + jnp.concatenate([uc[:, 1:, :], zy], axis=1)
            + jnp.concatenate([zz, uc[:, :, :-1]], axis=2)
            + jnp.concatenate([uc[:, :, 1:], zz], axis=2)
            - 6.0 * uc
        )
        out_ref[:, :, :] = v0

        def wait_halo(buf, rdir):
            rd = pltpu.make_async_remote_copy(
                src_ref=buf, dst_ref=buf,
                send_sem=send_sems.at[rdir], recv_sem=recv_sems.at[rdir],
                device_id=(mx, my, mz), device_id_type=pl.DeviceIdType.MESH,
            )
            rd.wait_recv()

        @pl.when(has_xlo)
        def _():
            wait_halo(hxlo, XLO)
            out_ref[0:1, :, :] = out_ref[0:1, :, :] + hxlo[:, :, :]

        @pl.when(has_xhi)
        def _():
            wait_halo(hxhi, XHI)
            out_ref[sx - 1:sx, :, :] = out_ref[sx - 1:sx, :, :] + hxhi[:, :, :]

        @pl.when(has_ylo)
        def _():
            wait_halo(hylo, YLO)
            out_ref[:, 0:1, :] = out_ref[:, 0:1, :] + hylo[:, :, :]

        @pl.when(has_yhi)
        def _():
            wait_halo(hyhi, YHI)
            out_ref[:, sy - 1:sy, :] = out_ref[:, sy - 1:sy, :] + hyhi[:, :, :]

        @pl.when(has_zlo)
        def _():
            wait_halo(hzlo, ZLO)
            out_ref[:, :, 0:1] = out_ref[:, :, 0:1] + hzlo[:, :, :]

        @pl.when(has_zhi)
        def _():
            wait_halo(hzhi, ZHI)
            out_ref[:, :, sz - 1:sz] = out_ref[:, :, sz - 1:sz] + hzhi[:, :, :]

        @pl.when(jnp.logical_not(has_xlo))
        def _():
            out_ref[0:1, :, :] = zx

        @pl.when(jnp.logical_not(has_xhi))
        def _():
            out_ref[sx - 1:sx, :, :] = zx

        @pl.when(jnp.logical_not(has_ylo))
        def _():
            out_ref[:, 0:1, :] = zy

        @pl.when(jnp.logical_not(has_yhi))
        def _():
            out_ref[:, sy - 1:sy, :] = zy

        @pl.when(jnp.logical_not(has_zlo))
        def _():
            out_ref[:, :, 0:1] = zz

        @pl.when(jnp.logical_not(has_zhi))
        def _():
            out_ref[:, :, sz - 1:sz] = zz

        def wait_sent(src, sdir):
            rd = pltpu.make_async_remote_copy(
                src_ref=src, dst_ref=src,
                send_sem=send_sems.at[sdir], recv_sem=recv_sems.at[sdir],
                device_id=(mx, my, mz), device_id_type=pl.DeviceIdType.MESH,
            )
            rd.wait_send()

        for cond, src, sdir in (
            (has_xlo, hxlo, XLO),
            (has_xhi, hxhi, XHI),
            (has_ylo, hylo, YLO),
            (has_yhi, hyhi, YHI),
            (has_zlo, hzlo, ZLO),
            (has_zhi, hzhi, ZHI),
        ):
            @pl.when(cond)
            def _(src=src, sdir=sdir):
                wait_sent(src, sdir)

    face = lambda shape: pltpu.VMEM(shape, jnp.float32)
    return pl.pallas_call(
        body,
        out_shape=jax.ShapeDtypeStruct((sx, sy, sz), jnp.float32),
        in_specs=[pl.BlockSpec(memory_space=pltpu.VMEM)],
        out_specs=pl.BlockSpec(memory_space=pltpu.VMEM),
        scratch_shapes=[
            face((sx, sy, 1)), face((sx, sy, 1)),
            face((1, sy, sz)), face((1, sy, sz)),
            face((sx, 1, sz)), face((sx, 1, sz)),
            face((sx, sy, 1)), face((sx, sy, 1)),
            pltpu.SemaphoreType.DMA((6,)),
            pltpu.SemaphoreType.DMA((6,)),
        ],
        compiler_params=pltpu.CompilerParams(collective_id=0),
    )(u)
